# Initial kernel scaffold; baseline (speedup 1.0000x reference)
import jax
import jax.numpy as jnp
from jax.experimental import pallas as pl


def kernel(x, edge_index, edge_attr, batch, ee_w1, ee_b1, ee_w2, ee_b2, conv1_w, conv1_b, stem_w, stem_b, lin_w, lin_b):
    raise NotImplementedError("write your pallas kernel here")


# speedup vs baseline: 8.0233x; 8.0233x over previous
import functools

import jax
import jax.numpy as jnp
from jax import lax
from jax.experimental import pallas as pl
from jax.experimental.pallas import tpu as pltpu
from jax.experimental.pallas import tpu_sc as plsc

N = 10000
E = 320000
D = 128
H = 128
C = 64
G = 16

NC = 2
NS = 16
NW = NC * NS
CH = 128
EPW = 10240
NCHK = EPW // CH
E_PAD = NW * EPW

_MESH = plsc.VectorSubcoreMesh(
    core_axis_name="c", subcore_axis_name="s", num_cores=NC, num_subcores=NS)


def _ew_body(srcf_ref, dstf_ref, attr_ref, p_ref, out_ref):
  s = srcf_ref[...]
  d = dstf_ref[...]
  t = attr_ref[...]
  h = jnp.maximum(s * p_ref[0] + d * p_ref[1] + t * p_ref[2] + p_ref[3], 0.0)
  w = jax.nn.sigmoid(h * p_ref[4] + p_ref[5])
  r = lax.broadcasted_iota(jnp.int32, w.shape, 0)
  out_ref[...] = jnp.where(r < E // 128, w, 0.0)


def _edge_weights(srcf, dstf, attr, params):
  out = pl.pallas_call(
      _ew_body,
      out_shape=jax.ShapeDtypeStruct((E_PAD // 128, 128), jnp.float32),
      in_specs=[
          pl.BlockSpec(memory_space=pltpu.VMEM),
          pl.BlockSpec(memory_space=pltpu.VMEM),
          pl.BlockSpec(memory_space=pltpu.VMEM),
          pl.BlockSpec(memory_space=pltpu.SMEM),
      ],
      out_specs=pl.BlockSpec(memory_space=pltpu.VMEM),
  )(srcf, dstf, attr, params)
  return out


def _dis_from(degp_ref):
  deg = degp_ref[:, 0:1] + degp_ref[:, 1:2] + 1.0
  return lax.rsqrt(jnp.maximum(deg, 1e-12))


def _h1_body(degp_ref, x_ref, w_ref, out_ref):
  dis = _dis_from(degp_ref)
  out_ref[...] = dis * jnp.dot(x_ref[...], w_ref[...],
                               preferred_element_type=jnp.float32)


def _h2_body(degp_ref, a0_ref, a1_ref, ht_ref, w_ref, b_ref, out_ref):
  dis = _dis_from(degp_ref)
  z = jnp.maximum(dis * (a0_ref[...] + a1_ref[...] + ht_ref[...]) + b_ref[...],
                  0.0)
  out_ref[...] = dis * jnp.dot(z, w_ref[...],
                               preferred_element_type=jnp.float32)


def _out_body(degp_ref, a0_ref, a1_ref, ht_ref, b_ref, batch_ref, lw_ref,
              lb_ref, out_ref):
  dis = _dis_from(degp_ref)
  z = jnp.maximum(dis * (a0_ref[...] + a1_ref[...] + ht_ref[...]) + b_ref[...],
                  0.0)
  gids = lax.broadcasted_iota(jnp.int32, (1, G), 1)
  onehot = (batch_ref[...] == gids).astype(jnp.float32)
  ze = jnp.concatenate([z, jnp.ones((N, 1), jnp.float32)], axis=1)
  sums = lax.dot_general(onehot, ze, (((0,), (0,)), ((), ())),
                         preferred_element_type=jnp.float32)
  pooled = sums[:, :H] / jnp.maximum(sums[:, H:H + 1], 1.0)
  out_ref[...] = jnp.dot(pooled, lw_ref[...],
                         preferred_element_type=jnp.float32) + lb_ref[...]


def _vmem_call(body, out_shape, *args):
  return pl.pallas_call(
      body,
      out_shape=out_shape,
      in_specs=[pl.BlockSpec(memory_space=pltpu.VMEM) for _ in args],
      out_specs=pl.BlockSpec(memory_space=pltpu.VMEM),
  )(*args)


def _deg_body(dst_hbm, ew_hbm, zeros_hbm, out_hbm, idx_v, w_v, acc_sh, sem):
  cid = lax.axis_index("c")
  sid = lax.axis_index("s")
  wid = sid * NC + cid
  pltpu.sync_copy(dst_hbm.at[wid], idx_v)
  pltpu.sync_copy(ew_hbm.at[wid], w_v)

  @pl.when(sid == 0)
  def _():
    pltpu.sync_copy(zeros_hbm, acc_sh)

  plsc.subcore_barrier()

  def chunk(j, carry):
    pltpu.sync_copy(w_v.at[j], acc_sh.at[idx_v.at[j]], add=True)
    return carry

  lax.fori_loop(0, NCHK, chunk, 0)
  plsc.subcore_barrier()

  @pl.when(sid == 0)
  def _():
    pltpu.sync_copy(acc_sh, out_hbm.at[cid])


@functools.partial(
    pl.kernel,
    out_type=jax.ShapeDtypeStruct((NC, N), jnp.float32),
    mesh=_MESH,
    scratch_types=[
        pltpu.VMEM((NCHK, CH), jnp.int32),
        pltpu.VMEM((NCHK, CH), jnp.float32),
        pltpu.VMEM_SHARED((N,), jnp.float32),
        pltpu.SemaphoreType.DMA,
    ],
)
def _deg_kernel(dst_hbm, ew_hbm, zeros_hbm, out_hbm, idx_v, w_v, acc_sh, sem):
  _deg_body(dst_hbm, ew_hbm, zeros_hbm, out_hbm, idx_v, w_v, acc_sh, sem)


def _agg_body(src_hbm, dst_hbm, ew_hbm, ht_hbm, zeros_hbm, out_hbm,
              src_v, dst_v, ew_v, buf, acc_sh, sem):
  cid = lax.axis_index("c")
  sid = lax.axis_index("s")
  wid = sid * NC + cid
  pltpu.sync_copy(src_hbm.at[wid], src_v)
  pltpu.sync_copy(dst_hbm.at[wid], dst_v)
  pltpu.sync_copy(ew_hbm.at[wid], ew_v)

  @pl.when(sid == 0)
  def _():
    pltpu.sync_copy(zeros_hbm, acc_sh)

  plsc.subcore_barrier()

  def chunk(j, carry):
    pltpu.async_copy(ht_hbm.at[src_v.at[j]], buf, sem).wait()

    def scale(g, c2):
      ew16 = ew_v[j, pl.ds(g * 16, 16)]
      for k in range(16):
        s = ew16[k]
        e = g * 16 + k
        for q in range(H // 16):
          sl = pl.ds(q * 16, 16)
          buf[e, sl] = buf[e, sl] * s
      return c2

    lax.fori_loop(0, CH // 16, scale, 0)
    pltpu.sync_copy(buf, acc_sh.at[dst_v.at[j]], add=True)
    return carry

  lax.fori_loop(0, NCHK, chunk, 0)
  plsc.subcore_barrier()

  @pl.when(sid == 0)
  def _():
    pltpu.sync_copy(acc_sh, out_hbm.at[cid])


@functools.partial(
    pl.kernel,
    out_type=jax.ShapeDtypeStruct((NC, N, H), jnp.float32),
    mesh=_MESH,
    scratch_types=[
        pltpu.VMEM((NCHK, CH), jnp.int32),
        pltpu.VMEM((NCHK, CH), jnp.int32),
        pltpu.VMEM((NCHK, CH), jnp.float32),
        pltpu.VMEM((CH, H), jnp.float32),
        pltpu.VMEM_SHARED((N, H), jnp.float32),
        pltpu.SemaphoreType.DMA,
    ],
)
def _agg_kernel(src_hbm, dst_hbm, ew_hbm, ht_hbm, zeros_hbm, out_hbm,
                src_v, dst_v, ew_v, buf, acc_sh, sem):
  _agg_body(src_hbm, dst_hbm, ew_hbm, ht_hbm, zeros_hbm, out_hbm,
            src_v, dst_v, ew_v, buf, acc_sh, sem)


def kernel(x, edge_index, edge_attr, batch, ee_w1, ee_b1, ee_w2, ee_b2,
           conv1_w, conv1_b, stem_w, stem_b, lin_w, lin_b):
  src = edge_index[0]
  dst = edge_index[1]
  pad = E_PAD - E
  src_p = jnp.pad(src, (0, pad))
  dst_p = jnp.pad(dst, (0, pad))
  attr_p = jnp.pad(edge_attr[:, 0], (0, pad))

  params = jnp.stack([
      ee_w1[0, 0], ee_w1[1, 0], ee_w1[2, 0], ee_b1[0], ee_w2[0, 0], ee_b2[0],
  ])

  srcf = src_p.astype(jnp.float32).reshape(E_PAD // 128, 128)
  dstf = dst_p.astype(jnp.float32).reshape(E_PAD // 128, 128)
  attrf = attr_p.reshape(E_PAD // 128, 128)
  ew = _edge_weights(srcf, dstf, attrf, params)

  src3 = src_p.reshape(NW, NCHK, CH)
  dst3 = dst_p.reshape(NW, NCHK, CH)
  ew3 = ew.reshape(NW, NCHK, CH)

  zeros_n = jnp.zeros((N,), jnp.float32)
  zeros_nh = jnp.zeros((N, H), jnp.float32)

  degp = _deg_kernel(dst3, ew3, zeros_n)
  degp_t = degp.T

  ht1 = _vmem_call(_h1_body, jax.ShapeDtypeStruct((N, H), jnp.float32),
                   degp_t, x, conv1_w)
  agg1 = _agg_kernel(src3, dst3, ew3, ht1, zeros_nh)

  ht2 = _vmem_call(_h2_body, jax.ShapeDtypeStruct((N, H), jnp.float32),
                   degp_t, agg1[0], agg1[1], ht1, stem_w,
                   conv1_b.reshape(1, H))
  agg2 = _agg_kernel(src3, dst3, ew3, ht2, zeros_nh)

  out = _vmem_call(_out_body, jax.ShapeDtypeStruct((G, C), jnp.float32),
                   degp_t, agg2[0], agg2[1], ht2, stem_b.reshape(1, H),
                   batch.reshape(N, 1), lin_w, lin_b.reshape(1, C))
  return out


# speedup vs baseline: 8.5728x; 1.0685x over previous
import functools

import jax
import jax.numpy as jnp
from jax import lax
from jax.experimental import pallas as pl
from jax.experimental.pallas import tpu as pltpu
from jax.experimental.pallas import tpu_sc as plsc

N = 10000
E = 320000
D = 128
H = 128
C = 64
G = 16

NC = 2
NS = 16
NW = NC * NS
CH = 64
EPW = 10240
NCHK = EPW // CH
SCCH = 16
NSC = NCHK // SCCH
E_PAD = NW * EPW

_MESH = plsc.VectorSubcoreMesh(
    core_axis_name="c", subcore_axis_name="s", num_cores=NC, num_subcores=NS)


def _ew_body(srcf_ref, dstf_ref, attr_ref, p_ref, out_ref):
  s = srcf_ref[...]
  d = dstf_ref[...]
  t = attr_ref[...]
  h = jnp.maximum(s * p_ref[0] + d * p_ref[1] + t * p_ref[2] + p_ref[3], 0.0)
  w = jax.nn.sigmoid(h * p_ref[4] + p_ref[5])
  r = lax.broadcasted_iota(jnp.int32, w.shape, 0)
  out_ref[...] = jnp.where(r < E // 128, w, 0.0)


def _edge_weights(srcf, dstf, attr, params):
  out = pl.pallas_call(
      _ew_body,
      out_shape=jax.ShapeDtypeStruct((E_PAD // 128, 128), jnp.float32),
      in_specs=[
          pl.BlockSpec(memory_space=pltpu.VMEM),
          pl.BlockSpec(memory_space=pltpu.VMEM),
          pl.BlockSpec(memory_space=pltpu.VMEM),
          pl.BlockSpec(memory_space=pltpu.SMEM),
      ],
      out_specs=pl.BlockSpec(memory_space=pltpu.VMEM),
  )(srcf, dstf, attr, params)
  return out


def _dis_from(degp_ref):
  deg = degp_ref[:, 0:1] + degp_ref[:, 1:2] + 1.0
  return lax.rsqrt(jnp.maximum(deg, 1e-12))


def _h1_body(degp_ref, x_ref, w_ref, out_ref):
  dis = _dis_from(degp_ref)
  out_ref[...] = dis * jnp.dot(x_ref[...], w_ref[...],
                               preferred_element_type=jnp.float32)


def _h2_body(degp_ref, a0_ref, a1_ref, ht_ref, w_ref, b_ref, out_ref):
  dis = _dis_from(degp_ref)
  z = jnp.maximum(dis * (a0_ref[...] + a1_ref[...] + ht_ref[...]) + b_ref[...],
                  0.0)
  out_ref[...] = dis * jnp.dot(z, w_ref[...],
                               preferred_element_type=jnp.float32)


def _out_body(degp_ref, a0_ref, a1_ref, ht_ref, b_ref, batch_ref, lw_ref,
              lb_ref, out_ref):
  dis = _dis_from(degp_ref)
  z = jnp.maximum(dis * (a0_ref[...] + a1_ref[...] + ht_ref[...]) + b_ref[...],
                  0.0)
  gids = lax.broadcasted_iota(jnp.int32, (1, G), 1)
  onehot = (batch_ref[...] == gids).astype(jnp.float32)
  ze = jnp.concatenate([z, jnp.ones((N, 1), jnp.float32)], axis=1)
  sums = lax.dot_general(onehot, ze, (((0,), (0,)), ((), ())),
                         preferred_element_type=jnp.float32)
  pooled = sums[:, :H] / jnp.maximum(sums[:, H:H + 1], 1.0)
  out_ref[...] = jnp.dot(pooled, lw_ref[...],
                         preferred_element_type=jnp.float32) + lb_ref[...]


def _vmem_call(body, out_shape, *args):
  return pl.pallas_call(
      body,
      out_shape=out_shape,
      in_specs=[pl.BlockSpec(memory_space=pltpu.VMEM) for _ in args],
      out_specs=pl.BlockSpec(memory_space=pltpu.VMEM),
  )(*args)


def _deg_body(dst_hbm, ew_hbm, zeros_hbm, out_hbm, idx_v, w_v, acc_sh, sem):
  cid = lax.axis_index("c")
  sid = lax.axis_index("s")
  wid = sid * NC + cid
  pltpu.sync_copy(dst_hbm.at[wid], idx_v)
  pltpu.sync_copy(ew_hbm.at[wid], w_v)

  @pl.when(sid == 0)
  def _():
    pltpu.sync_copy(zeros_hbm, acc_sh)

  plsc.subcore_barrier()

  def chunk(j, carry):
    pltpu.sync_copy(w_v.at[j], acc_sh.at[idx_v.at[j]], add=True)
    return carry

  lax.fori_loop(0, NCHK, chunk, 0)
  plsc.subcore_barrier()

  @pl.when(sid == 0)
  def _():
    pltpu.sync_copy(acc_sh, out_hbm.at[cid])


@functools.partial(
    pl.kernel,
    out_type=jax.ShapeDtypeStruct((NC, N), jnp.float32),
    mesh=_MESH,
    scratch_types=[
        pltpu.VMEM((NCHK, CH), jnp.int32),
        pltpu.VMEM((NCHK, CH), jnp.float32),
        pltpu.VMEM_SHARED((N,), jnp.float32),
        pltpu.SemaphoreType.DMA,
    ],
)
def _deg_kernel(dst_hbm, ew_hbm, zeros_hbm, out_hbm, idx_v, w_v, acc_sh, sem):
  _deg_body(dst_hbm, ew_hbm, zeros_hbm, out_hbm, idx_v, w_v, acc_sh, sem)


def _agg_body(src_hbm, dst_hbm, ew_hbm, ht_hbm, zeros_hbm, out_hbm,
              src_v, dst_v, ew_v, buf0, buf1, acc_sh, sem0, sem1):
  cid = lax.axis_index("c")
  sid = lax.axis_index("s")
  wid = sid * NC + cid

  @pl.when(sid == 0)
  def _():
    pltpu.sync_copy(zeros_hbm, acc_sh)

  plsc.subcore_barrier()

  def scale_scatter(j, buf):
    def scale(g, c2):
      ew16 = ew_v[j, pl.ds(g * 16, 16)]
      for k in range(16):
        s = ew16[k]
        e = g * 16 + k
        for q in range(H // 16):
          sl = pl.ds(q * 16, 16)
          buf[e, sl] = buf[e, sl] * s
      return c2

    lax.fori_loop(0, CH // 16, scale, 0)
    pltpu.sync_copy(buf, acc_sh.at[dst_v.at[j]], add=True)

  def super_body(s, carry):
    pltpu.sync_copy(src_hbm.at[wid, s], src_v)
    pltpu.sync_copy(dst_hbm.at[wid, s], dst_v)
    pltpu.sync_copy(ew_hbm.at[wid, s], ew_v)
    pltpu.async_copy(ht_hbm.at[src_v.at[0]], buf0, sem0)

    def chunk2(jj, c2):
      j0 = 2 * jj
      j1 = j0 + 1
      pltpu.async_copy(ht_hbm.at[src_v.at[j1]], buf1, sem1)
      pltpu.make_async_copy(ht_hbm.at[src_v.at[j0]], buf0, sem0).wait()
      scale_scatter(j0, buf0)

      @pl.when(j0 + 2 < SCCH)
      def _():
        pltpu.async_copy(ht_hbm.at[src_v.at[j0 + 2]], buf0, sem0)

      pltpu.make_async_copy(ht_hbm.at[src_v.at[j1]], buf1, sem1).wait()
      scale_scatter(j1, buf1)
      return c2

    lax.fori_loop(0, SCCH // 2, chunk2, 0)
    return carry

  lax.fori_loop(0, NSC, super_body, 0)
  plsc.subcore_barrier()

  @pl.when(sid == 0)
  def _():
    pltpu.sync_copy(acc_sh, out_hbm.at[cid])


@functools.partial(
    pl.kernel,
    out_type=jax.ShapeDtypeStruct((NC, N, H), jnp.float32),
    mesh=_MESH,
    scratch_types=[
        pltpu.VMEM((SCCH, CH), jnp.int32),
        pltpu.VMEM((SCCH, CH), jnp.int32),
        pltpu.VMEM((SCCH, CH), jnp.float32),
        pltpu.VMEM((CH, H), jnp.float32),
        pltpu.VMEM((CH, H), jnp.float32),
        pltpu.VMEM_SHARED((N, H), jnp.float32),
        pltpu.SemaphoreType.DMA,
        pltpu.SemaphoreType.DMA,
    ],
)
def _agg_kernel(src_hbm, dst_hbm, ew_hbm, ht_hbm, zeros_hbm, out_hbm,
                src_v, dst_v, ew_v, buf0, buf1, acc_sh, sem0, sem1):
  _agg_body(src_hbm, dst_hbm, ew_hbm, ht_hbm, zeros_hbm, out_hbm,
            src_v, dst_v, ew_v, buf0, buf1, acc_sh, sem0, sem1)


def kernel(x, edge_index, edge_attr, batch, ee_w1, ee_b1, ee_w2, ee_b2,
           conv1_w, conv1_b, stem_w, stem_b, lin_w, lin_b):
  src = edge_index[0]
  dst = edge_index[1]
  pad = E_PAD - E
  src_p = jnp.pad(src, (0, pad))
  dst_p = jnp.pad(dst, (0, pad))
  attr_p = jnp.pad(edge_attr[:, 0], (0, pad))

  params = jnp.stack([
      ee_w1[0, 0], ee_w1[1, 0], ee_w1[2, 0], ee_b1[0], ee_w2[0, 0], ee_b2[0],
  ])

  srcf = src_p.astype(jnp.float32).reshape(E_PAD // 128, 128)
  dstf = dst_p.astype(jnp.float32).reshape(E_PAD // 128, 128)
  attrf = attr_p.reshape(E_PAD // 128, 128)
  ew = _edge_weights(srcf, dstf, attrf, params)

  src3 = src_p.reshape(NW, NCHK, CH)
  dst3 = dst_p.reshape(NW, NCHK, CH)
  ew3 = ew.reshape(NW, NCHK, CH)
  src4 = src_p.reshape(NW, NSC, SCCH, CH)
  dst4 = dst_p.reshape(NW, NSC, SCCH, CH)
  ew4 = ew.reshape(NW, NSC, SCCH, CH)

  zeros_n = jnp.zeros((N,), jnp.float32)
  zeros_nh = jnp.zeros((N, H), jnp.float32)

  degp = _deg_kernel(dst3, ew3, zeros_n)
  degp_t = degp.T

  ht1 = _vmem_call(_h1_body, jax.ShapeDtypeStruct((N, H), jnp.float32),
                   degp_t, x, conv1_w)
  agg1 = _agg_kernel(src4, dst4, ew4, ht1, zeros_nh)

  ht2 = _vmem_call(_h2_body, jax.ShapeDtypeStruct((N, H), jnp.float32),
                   degp_t, agg1[0], agg1[1], ht1, stem_w,
                   conv1_b.reshape(1, H))
  agg2 = _agg_kernel(src4, dst4, ew4, ht2, zeros_nh)

  out = _vmem_call(_out_body, jax.ShapeDtypeStruct((G, C), jnp.float32),
                   degp_t, agg2[0], agg2[1], ht2, stem_b.reshape(1, H),
                   batch.reshape(N, 1), lin_w, lin_b.reshape(1, C))
  return out


# speedup vs baseline: 11.5695x; 1.3496x over previous
import functools

import jax
import jax.numpy as jnp
import numpy as np
from jax import lax
from jax.experimental import pallas as pl
from jax.experimental.pallas import tpu as pltpu
from jax.experimental.pallas import tpu_sc as plsc

N = 10000
E = 320000
D = 128
H = 128
C = 64
G = 16

NC = 2
NS = 16
NW = NC * NS
CH = 64
EPW = 10240
NCHK = EPW // CH
SCCH = 16
NSC = NCHK // SCCH
E_PAD = NW * EPW

_MESH = plsc.VectorSubcoreMesh(
    core_axis_name="c", subcore_axis_name="s", num_cores=NC, num_subcores=NS)

_PERM = np.zeros((H,), np.int64)
for _q in range(H // 32):
  for _k in range(16):
    _PERM[32 * _q + _k] = 32 * _q + 2 * _k
    _PERM[32 * _q + 16 + _k] = 32 * _q + 2 * _k + 1
_PERM_M = np.zeros((H, H), np.float32)
_PERM_M[np.arange(H), _PERM] = 1.0


def _ew_body(srcf_ref, dstf_ref, attr_ref, p_ref, out_ref):
  s = srcf_ref[...]
  d = dstf_ref[...]
  t = attr_ref[...]
  h = jnp.maximum(s * p_ref[0] + d * p_ref[1] + t * p_ref[2] + p_ref[3], 0.0)
  w = jax.nn.sigmoid(h * p_ref[4] + p_ref[5])
  r = lax.broadcasted_iota(jnp.int32, w.shape, 0)
  out_ref[...] = jnp.where(r < E // 128, w, 0.0)


def _edge_weights(srcf, dstf, attr, params):
  out = pl.pallas_call(
      _ew_body,
      out_shape=jax.ShapeDtypeStruct((E_PAD // 128, 128), jnp.float32),
      in_specs=[
          pl.BlockSpec(memory_space=pltpu.VMEM),
          pl.BlockSpec(memory_space=pltpu.VMEM),
          pl.BlockSpec(memory_space=pltpu.VMEM),
          pl.BlockSpec(memory_space=pltpu.SMEM),
      ],
      out_specs=pl.BlockSpec(memory_space=pltpu.VMEM),
  )(srcf, dstf, attr, params)
  return out


def _dis_from(degp_ref):
  deg = degp_ref[:, 0:1] + degp_ref[:, 1:2] + 1.0
  return lax.rsqrt(jnp.maximum(deg, 1e-12))


def _h1_body(degp_ref, x_ref, w_ref, out_ref):
  dis = _dis_from(degp_ref)
  out_ref[...] = dis * jnp.dot(x_ref[...], w_ref[...],
                               preferred_element_type=jnp.float32)


def _h2_body(degp_ref, a0_ref, a1_ref, ht_ref, w_ref, b_ref, m_ref, out_ref):
  dis = _dis_from(degp_ref)
  agg = jnp.dot(a0_ref[...] + a1_ref[...], m_ref[...],
                preferred_element_type=jnp.float32)
  z = jnp.maximum(dis * (agg + ht_ref[...]) + b_ref[...], 0.0)
  out_ref[...] = dis * jnp.dot(z, w_ref[...],
                               preferred_element_type=jnp.float32)


def _out_body(degp_ref, a0_ref, a1_ref, ht_ref, b_ref, batch_ref, lw_ref,
              lb_ref, m_ref, out_ref):
  dis = _dis_from(degp_ref)
  agg = jnp.dot(a0_ref[...] + a1_ref[...], m_ref[...],
                preferred_element_type=jnp.float32)
  z = jnp.maximum(dis * (agg + ht_ref[...]) + b_ref[...], 0.0)
  gids = lax.broadcasted_iota(jnp.int32, (1, G), 1)
  onehot = (batch_ref[...] == gids).astype(jnp.float32)
  ze = jnp.concatenate([z, jnp.ones((N, 1), jnp.float32)], axis=1)
  sums = lax.dot_general(onehot, ze, (((0,), (0,)), ((), ())),
                         preferred_element_type=jnp.float32)
  pooled = sums[:, :H] / jnp.maximum(sums[:, H:H + 1], 1.0)
  out_ref[...] = jnp.dot(pooled, lw_ref[...],
                         preferred_element_type=jnp.float32) + lb_ref[...]


def _vmem_call(body, out_shape, *args):
  return pl.pallas_call(
      body,
      out_shape=out_shape,
      in_specs=[pl.BlockSpec(memory_space=pltpu.VMEM) for _ in args],
      out_specs=pl.BlockSpec(memory_space=pltpu.VMEM),
  )(*args)


def _deg_body(dst_hbm, ew_hbm, zeros_hbm, out_hbm, idx_v, w_v, acc_sh, sem):
  cid = lax.axis_index("c")
  sid = lax.axis_index("s")
  wid = sid * NC + cid
  pltpu.sync_copy(dst_hbm.at[wid], idx_v)
  pltpu.sync_copy(ew_hbm.at[wid], w_v)

  @pl.when(sid == 0)
  def _():
    pltpu.sync_copy(zeros_hbm, acc_sh)

  plsc.subcore_barrier()

  def chunk(j, carry):
    pltpu.sync_copy(w_v.at[j], acc_sh.at[idx_v.at[j]], add=True)
    return carry

  lax.fori_loop(0, NCHK, chunk, 0)
  plsc.subcore_barrier()

  @pl.when(sid == 0)
  def _():
    pltpu.sync_copy(acc_sh, out_hbm.at[cid])


@functools.partial(
    pl.kernel,
    out_type=jax.ShapeDtypeStruct((NC, N), jnp.float32),
    mesh=_MESH,
    scratch_types=[
        pltpu.VMEM((NCHK, CH), jnp.int32),
        pltpu.VMEM((NCHK, CH), jnp.float32),
        pltpu.VMEM_SHARED((N,), jnp.float32),
        pltpu.SemaphoreType.DMA,
    ],
)
def _deg_kernel(dst_hbm, ew_hbm, zeros_hbm, out_hbm, idx_v, w_v, acc_sh, sem):
  _deg_body(dst_hbm, ew_hbm, zeros_hbm, out_hbm, idx_v, w_v, acc_sh, sem)


def _agg_body(src_hbm, dst_hbm, ew_hbm, ht_hbm, zeros_hbm, out_hbm,
              src_v, dst_v, ew_v, buf0, buf1, sb0, sb1, acc_sh,
              sem0, sem1, ssem0, ssem1):
  cid = lax.axis_index("c")
  sid = lax.axis_index("s")
  wid = sid * NC + cid

  @pl.when(sid == 0)
  def _():
    pltpu.sync_copy(zeros_hbm, acc_sh)

  plsc.subcore_barrier()

  def unpack_scale(j, buf, sb):
    def scale(g, c2):
      ew16 = ew_v[j, pl.ds(g * 16, 16)]
      for k in range(16):
        s = ew16[k]
        e = g * 16 + k
        for q in range(H // 32):
          w16 = buf[e, pl.ds(q * 16, 16)]
          w32 = plsc.bitcast(w16, jnp.bfloat16)
          lo, hi = plsc.unpack(w32, format=plsc.PackFormat.INTERLEAVED)
          sb[e, pl.ds(q * 32, 16)] = lo * s
          sb[e, pl.ds(q * 32 + 16, 16)] = hi * s
      return c2

    lax.fori_loop(0, CH // 16, scale, 0)

  def super_body(s, carry):
    pltpu.sync_copy(src_hbm.at[wid, s], src_v)
    pltpu.sync_copy(dst_hbm.at[wid, s], dst_v)
    pltpu.sync_copy(ew_hbm.at[wid, s], ew_v)
    pltpu.async_copy(ht_hbm.at[src_v.at[0]], buf0, sem0)

    def chunk2(jj, c2):
      j0 = 2 * jj
      j1 = j0 + 1
      pltpu.async_copy(ht_hbm.at[src_v.at[j1]], buf1, sem1)
      pltpu.make_async_copy(ht_hbm.at[src_v.at[j0]], buf0, sem0).wait()

      @pl.when(j0 >= 2)
      def _():
        pltpu.make_async_copy(sb0, acc_sh.at[dst_v.at[j0 - 2]], ssem0).wait()

      unpack_scale(j0, buf0, sb0)
      pltpu.async_copy(sb0, acc_sh.at[dst_v.at[j0]], ssem0, add=True)

      @pl.when(j0 + 2 < SCCH)
      def _():
        pltpu.async_copy(ht_hbm.at[src_v.at[j0 + 2]], buf0, sem0)

      pltpu.make_async_copy(ht_hbm.at[src_v.at[j1]], buf1, sem1).wait()

      @pl.when(j1 >= 2)
      def _():
        pltpu.make_async_copy(sb1, acc_sh.at[dst_v.at[j1 - 2]], ssem1).wait()

      unpack_scale(j1, buf1, sb1)
      pltpu.async_copy(sb1, acc_sh.at[dst_v.at[j1]], ssem1, add=True)
      return c2

    lax.fori_loop(0, SCCH // 2, chunk2, 0)
    pltpu.make_async_copy(sb0, acc_sh.at[dst_v.at[SCCH - 2]], ssem0).wait()
    pltpu.make_async_copy(sb1, acc_sh.at[dst_v.at[SCCH - 1]], ssem1).wait()
    return carry

  lax.fori_loop(0, NSC, super_body, 0)
  plsc.subcore_barrier()

  @pl.when(sid == 0)
  def _():
    pltpu.sync_copy(acc_sh, out_hbm.at[cid])


@functools.partial(
    pl.kernel,
    out_type=jax.ShapeDtypeStruct((NC, N, H), jnp.float32),
    mesh=_MESH,
    scratch_types=[
        pltpu.VMEM((SCCH, CH), jnp.int32),
        pltpu.VMEM((SCCH, CH), jnp.int32),
        pltpu.VMEM((SCCH, CH), jnp.float32),
        pltpu.VMEM((CH, H // 2), jnp.int32),
        pltpu.VMEM((CH, H // 2), jnp.int32),
        pltpu.VMEM((CH, H), jnp.float32),
        pltpu.VMEM((CH, H), jnp.float32),
        pltpu.VMEM_SHARED((N, H), jnp.float32),
        pltpu.SemaphoreType.DMA,
        pltpu.SemaphoreType.DMA,
        pltpu.SemaphoreType.DMA,
        pltpu.SemaphoreType.DMA,
    ],
    compiler_params=pltpu.CompilerParams(
        needs_layout_passes=False, use_tc_tiling_on_sc=False),
)
def _agg_kernel(src_hbm, dst_hbm, ew_hbm, ht_hbm, zeros_hbm, out_hbm,
                src_v, dst_v, ew_v, buf0, buf1, sb0, sb1, acc_sh,
                sem0, sem1, ssem0, ssem1):
  _agg_body(src_hbm, dst_hbm, ew_hbm, ht_hbm, zeros_hbm, out_hbm,
            src_v, dst_v, ew_v, buf0, buf1, sb0, sb1, acc_sh,
            sem0, sem1, ssem0, ssem1)


def kernel(x, edge_index, edge_attr, batch, ee_w1, ee_b1, ee_w2, ee_b2,
           conv1_w, conv1_b, stem_w, stem_b, lin_w, lin_b):
  src = edge_index[0]
  dst = edge_index[1]
  pad = E_PAD - E
  src_p = jnp.pad(src, (0, pad))
  dst_p = jnp.pad(dst, (0, pad))
  attr_p = jnp.pad(edge_attr[:, 0], (0, pad))

  params = jnp.stack([
      ee_w1[0, 0], ee_w1[1, 0], ee_w1[2, 0], ee_b1[0], ee_w2[0, 0], ee_b2[0],
  ])

  srcf = src_p.astype(jnp.float32).reshape(E_PAD // 128, 128)
  dstf = dst_p.astype(jnp.float32).reshape(E_PAD // 128, 128)
  attrf = attr_p.reshape(E_PAD // 128, 128)
  ew = _edge_weights(srcf, dstf, attrf, params)

  src3 = src_p.reshape(NW, NCHK, CH)
  dst3 = dst_p.reshape(NW, NCHK, CH)
  ew3 = ew.reshape(NW, NCHK, CH)
  src4 = src_p.reshape(NW, NSC, SCCH, CH)
  dst4 = dst_p.reshape(NW, NSC, SCCH, CH)
  ew4 = ew.reshape(NW, NSC, SCCH, CH)

  zeros_n = jnp.zeros((N,), jnp.float32)
  zeros_nh = jnp.zeros((N, H), jnp.float32)

  degp = _deg_kernel(dst3, ew3, zeros_n)
  degp_t = degp.T
  perm_m = jnp.asarray(_PERM_M)

  def pack_bf16(ht):
    return lax.bitcast_convert_type(
        ht.astype(jnp.bfloat16).reshape(N, H // 2, 2), jnp.int32)

  ht1 = _vmem_call(_h1_body, jax.ShapeDtypeStruct((N, H), jnp.float32),
                   degp_t, x, conv1_w)
  agg1 = _agg_kernel(src4, dst4, ew4, pack_bf16(ht1), zeros_nh)

  ht2 = _vmem_call(_h2_body, jax.ShapeDtypeStruct((N, H), jnp.float32),
                   degp_t, agg1[0], agg1[1], ht1, stem_w,
                   conv1_b.reshape(1, H), perm_m)
  agg2 = _agg_kernel(src4, dst4, ew4, pack_bf16(ht2), zeros_nh)

  out = _vmem_call(_out_body, jax.ShapeDtypeStruct((G, C), jnp.float32),
                   degp_t, agg2[0], agg2[1], ht2, stem_b.reshape(1, H),
                   batch.reshape(N, 1), lin_w, lin_b.reshape(1, C), perm_m)
  return out


# speedup vs baseline: 12.2691x; 1.0605x over previous
import functools

import jax
import jax.numpy as jnp
import numpy as np
from jax import lax
from jax.experimental import pallas as pl
from jax.experimental.pallas import tpu as pltpu
from jax.experimental.pallas import tpu_sc as plsc

N = 10000
E = 320000
D = 128
H = 128
C = 64
G = 16

NC = 2
NS = 16
NW = NC * NS
CH = 64
EPW = 10240
NCHK = EPW // CH
SCCH = 16
NSC = NCHK // SCCH
TOTSC = NW * NSC
K0 = 11
K1 = 2 * NSC - K0
E_PAD = NW * EPW

_MESH = plsc.VectorSubcoreMesh(
    core_axis_name="c", subcore_axis_name="s", num_cores=NC, num_subcores=NS)

_PERM = np.zeros((H,), np.int64)
for _q in range(H // 32):
  for _k in range(16):
    _PERM[32 * _q + _k] = 32 * _q + 2 * _k
    _PERM[32 * _q + 16 + _k] = 32 * _q + 2 * _k + 1
_PERM_M = np.zeros((H, H), np.float32)
_PERM_M[np.arange(H), _PERM] = 1.0


def _ew_body(srcf_ref, dstf_ref, attr_ref, p_ref, out_ref):
  s = srcf_ref[...]
  d = dstf_ref[...]
  t = attr_ref[...]
  h = jnp.maximum(s * p_ref[0] + d * p_ref[1] + t * p_ref[2] + p_ref[3], 0.0)
  w = jax.nn.sigmoid(h * p_ref[4] + p_ref[5])
  r = lax.broadcasted_iota(jnp.int32, w.shape, 0)
  out_ref[...] = jnp.where(r < E // 128, w, 0.0)


def _edge_weights(srcf, dstf, attr, params):
  out = pl.pallas_call(
      _ew_body,
      out_shape=jax.ShapeDtypeStruct((E_PAD // 128, 128), jnp.float32),
      in_specs=[
          pl.BlockSpec(memory_space=pltpu.VMEM),
          pl.BlockSpec(memory_space=pltpu.VMEM),
          pl.BlockSpec(memory_space=pltpu.VMEM),
          pl.BlockSpec(memory_space=pltpu.SMEM),
      ],
      out_specs=pl.BlockSpec(memory_space=pltpu.VMEM),
  )(srcf, dstf, attr, params)
  return out


def _dis_from(degp_ref):
  deg = degp_ref[:, 0:1] + degp_ref[:, 1:2] + 1.0
  return lax.rsqrt(jnp.maximum(deg, 1e-12))


def _mm_body(x_ref, w_ref, out_ref):
  out_ref[...] = jnp.dot(x_ref[...], w_ref[...],
                         preferred_element_type=jnp.float32)


def _scale_body(degp_ref, h_ref, out_ref):
  out_ref[...] = _dis_from(degp_ref) * h_ref[...]


def _h2_body(degp_ref, a0_ref, a1_ref, ht_ref, w_ref, b_ref, m_ref, out_ref):
  dis = _dis_from(degp_ref)
  agg = jnp.dot(a0_ref[...] + a1_ref[...], m_ref[...],
                preferred_element_type=jnp.float32)
  z = jnp.maximum(dis * (agg + ht_ref[...]) + b_ref[...], 0.0)
  out_ref[...] = dis * jnp.dot(z, w_ref[...],
                               preferred_element_type=jnp.float32)


def _out_body(degp_ref, a0_ref, a1_ref, ht_ref, b_ref, batch_ref, lw_ref,
              lb_ref, m_ref, out_ref):
  dis = _dis_from(degp_ref)
  agg = jnp.dot(a0_ref[...] + a1_ref[...], m_ref[...],
                preferred_element_type=jnp.float32)
  z = jnp.maximum(dis * (agg + ht_ref[...]) + b_ref[...], 0.0)
  gids = lax.broadcasted_iota(jnp.int32, (1, G), 1)
  onehot = (batch_ref[...] == gids).astype(jnp.float32)
  ze = jnp.concatenate([z, jnp.ones((N, 1), jnp.float32)], axis=1)
  sums = lax.dot_general(onehot, ze, (((0,), (0,)), ((), ())),
                         preferred_element_type=jnp.float32)
  pooled = sums[:, :H] / jnp.maximum(sums[:, H:H + 1], 1.0)
  out_ref[...] = jnp.dot(pooled, lw_ref[...],
                         preferred_element_type=jnp.float32) + lb_ref[...]


def _vmem_call(body, out_shape, *args):
  return pl.pallas_call(
      body,
      out_shape=out_shape,
      in_specs=[pl.BlockSpec(memory_space=pltpu.VMEM) for _ in args],
      out_specs=pl.BlockSpec(memory_space=pltpu.VMEM),
  )(*args)


def _deg_body(dst_hbm, ew_hbm, zeros_hbm, out_hbm, idx_v, w_v, acc_sh, sem):
  cid = lax.axis_index("c")
  sid = lax.axis_index("s")
  wid = sid * NC + cid
  pltpu.sync_copy(dst_hbm.at[wid], idx_v)
  pltpu.sync_copy(ew_hbm.at[wid], w_v)

  @pl.when(sid == 0)
  def _():
    pltpu.sync_copy(zeros_hbm, acc_sh)

  plsc.subcore_barrier()

  def chunk(j, carry):
    pltpu.sync_copy(w_v.at[j], acc_sh.at[idx_v.at[j]], add=True)
    return carry

  lax.fori_loop(0, NCHK, chunk, 0)
  plsc.subcore_barrier()

  @pl.when(sid == 0)
  def _():
    pltpu.sync_copy(acc_sh, out_hbm.at[cid])


@functools.partial(
    pl.kernel,
    out_type=jax.ShapeDtypeStruct((NC, N), jnp.float32),
    mesh=_MESH,
    scratch_types=[
        pltpu.VMEM((NCHK, CH), jnp.int32),
        pltpu.VMEM((NCHK, CH), jnp.float32),
        pltpu.VMEM_SHARED((N,), jnp.float32),
        pltpu.SemaphoreType.DMA,
    ],
)
def _deg_kernel(dst_hbm, ew_hbm, zeros_hbm, out_hbm, idx_v, w_v, acc_sh, sem):
  _deg_body(dst_hbm, ew_hbm, zeros_hbm, out_hbm, idx_v, w_v, acc_sh, sem)


def _agg_body(src_hbm, dst_hbm, ew_hbm, ht_hbm, zeros_hbm, out_hbm,
              src_v, dst_v, ew_v, buf0, buf1, sb0, sb1, acc_sh,
              sem0, sem1, ssem0, ssem1):
  cid = lax.axis_index("c")
  sid = lax.axis_index("s")
  base = jnp.where(cid == 0, sid * K0, NS * K0 + sid * K1)
  nsc_local = jnp.where(cid == 0, K0, K1)

  @pl.when(sid == 0)
  def _():
    pltpu.sync_copy(zeros_hbm, acc_sh)

  plsc.subcore_barrier()

  def unpack_scale(j, buf, sb):
    def scale(g, c2):
      ew16 = ew_v[j, pl.ds(g * 16, 16)]
      for k in range(16):
        s = ew16[k]
        e = g * 16 + k
        for q in range(H // 32):
          w16 = buf[e, pl.ds(q * 16, 16)]
          w32 = plsc.bitcast(w16, jnp.bfloat16)
          lo, hi = plsc.unpack(w32, format=plsc.PackFormat.INTERLEAVED)
          sb[e, pl.ds(q * 32, 16)] = lo * s
          sb[e, pl.ds(q * 32 + 16, 16)] = hi * s
      return c2

    lax.fori_loop(0, CH // 16, scale, 0)

  def super_body(s, carry):
    pltpu.sync_copy(src_hbm.at[base + s], src_v)
    pltpu.sync_copy(dst_hbm.at[base + s], dst_v)
    pltpu.sync_copy(ew_hbm.at[base + s], ew_v)
    pltpu.async_copy(ht_hbm.at[src_v.at[0]], buf0, sem0)

    def chunk2(jj, c2):
      j0 = 2 * jj
      j1 = j0 + 1
      pltpu.async_copy(ht_hbm.at[src_v.at[j1]], buf1, sem1)
      pltpu.make_async_copy(ht_hbm.at[src_v.at[j0]], buf0, sem0).wait()

      @pl.when(j0 >= 2)
      def _():
        pltpu.make_async_copy(sb0, acc_sh.at[dst_v.at[j0 - 2]], ssem0).wait()

      unpack_scale(j0, buf0, sb0)
      pltpu.async_copy(sb0, acc_sh.at[dst_v.at[j0]], ssem0, add=True)

      @pl.when(j0 + 2 < SCCH)
      def _():
        pltpu.async_copy(ht_hbm.at[src_v.at[j0 + 2]], buf0, sem0)

      pltpu.make_async_copy(ht_hbm.at[src_v.at[j1]], buf1, sem1).wait()

      @pl.when(j1 >= 2)
      def _():
        pltpu.make_async_copy(sb1, acc_sh.at[dst_v.at[j1 - 2]], ssem1).wait()

      unpack_scale(j1, buf1, sb1)
      pltpu.async_copy(sb1, acc_sh.at[dst_v.at[j1]], ssem1, add=True)
      return c2

    lax.fori_loop(0, SCCH // 2, chunk2, 0)
    pltpu.make_async_copy(sb0, acc_sh.at[dst_v.at[SCCH - 2]], ssem0).wait()
    pltpu.make_async_copy(sb1, acc_sh.at[dst_v.at[SCCH - 1]], ssem1).wait()
    return carry

  lax.fori_loop(0, nsc_local, super_body, 0)
  plsc.subcore_barrier()

  @pl.when(sid == 0)
  def _():
    pltpu.sync_copy(acc_sh, out_hbm.at[cid])


@functools.partial(
    pl.kernel,
    out_type=jax.ShapeDtypeStruct((NC, N, H), jnp.float32),
    mesh=_MESH,
    scratch_types=[
        pltpu.VMEM((SCCH, CH), jnp.int32),
        pltpu.VMEM((SCCH, CH), jnp.int32),
        pltpu.VMEM((SCCH, CH), jnp.float32),
        pltpu.VMEM((CH, H // 2), jnp.int32),
        pltpu.VMEM((CH, H // 2), jnp.int32),
        pltpu.VMEM((CH, H), jnp.float32),
        pltpu.VMEM((CH, H), jnp.float32),
        pltpu.VMEM_SHARED((N, H), jnp.float32),
        pltpu.SemaphoreType.DMA,
        pltpu.SemaphoreType.DMA,
        pltpu.SemaphoreType.DMA,
        pltpu.SemaphoreType.DMA,
    ],
    compiler_params=pltpu.CompilerParams(
        needs_layout_passes=False, use_tc_tiling_on_sc=False),
)
def _agg_kernel(src_hbm, dst_hbm, ew_hbm, ht_hbm, zeros_hbm, out_hbm,
                src_v, dst_v, ew_v, buf0, buf1, sb0, sb1, acc_sh,
                sem0, sem1, ssem0, ssem1):
  _agg_body(src_hbm, dst_hbm, ew_hbm, ht_hbm, zeros_hbm, out_hbm,
            src_v, dst_v, ew_v, buf0, buf1, sb0, sb1, acc_sh,
            sem0, sem1, ssem0, ssem1)


def kernel(x, edge_index, edge_attr, batch, ee_w1, ee_b1, ee_w2, ee_b2,
           conv1_w, conv1_b, stem_w, stem_b, lin_w, lin_b):
  src = edge_index[0]
  dst = edge_index[1]
  pad = E_PAD - E
  src_p = jnp.pad(src, (0, pad))
  dst_p = jnp.pad(dst, (0, pad))
  attr_p = jnp.pad(edge_attr[:, 0], (0, pad))

  params = jnp.stack([
      ee_w1[0, 0], ee_w1[1, 0], ee_w1[2, 0], ee_b1[0], ee_w2[0, 0], ee_b2[0],
  ])

  srcf = src_p.astype(jnp.float32).reshape(E_PAD // 128, 128)
  dstf = dst_p.astype(jnp.float32).reshape(E_PAD // 128, 128)
  attrf = attr_p.reshape(E_PAD // 128, 128)
  ew = _edge_weights(srcf, dstf, attrf, params)

  src3 = src_p.reshape(NW, NCHK, CH)
  dst3 = dst_p.reshape(NW, NCHK, CH)
  ew3 = ew.reshape(NW, NCHK, CH)
  src4 = src_p.reshape(TOTSC, SCCH, CH)
  dst4 = dst_p.reshape(TOTSC, SCCH, CH)
  ew4 = ew.reshape(TOTSC, SCCH, CH)

  zeros_n = jnp.zeros((N,), jnp.float32)
  zeros_nh = jnp.zeros((N, H), jnp.float32)

  degp = _deg_kernel(dst3, ew3, zeros_n)
  degp_t = degp.T
  perm_m = jnp.asarray(_PERM_M)

  def pack_bf16(ht):
    return lax.bitcast_convert_type(
        ht.astype(jnp.bfloat16).reshape(N, H // 2, 2), jnp.int32)

  h1 = _vmem_call(_mm_body, jax.ShapeDtypeStruct((N, H), jnp.float32),
                  x, conv1_w)
  ht1 = _vmem_call(_scale_body, jax.ShapeDtypeStruct((N, H), jnp.float32),
                   degp_t, h1)
  agg1 = _agg_kernel(src4, dst4, ew4, pack_bf16(ht1), zeros_nh)

  ht2 = _vmem_call(_h2_body, jax.ShapeDtypeStruct((N, H), jnp.float32),
                   degp_t, agg1[0], agg1[1], ht1, stem_w,
                   conv1_b.reshape(1, H), perm_m)
  agg2 = _agg_kernel(src4, dst4, ew4, pack_bf16(ht2), zeros_nh)

  out = _vmem_call(_out_body, jax.ShapeDtypeStruct((G, C), jnp.float32),
                   degp_t, agg2[0], agg2[1], ht2, stem_b.reshape(1, H),
                   batch.reshape(N, 1), lin_w, lin_b.reshape(1, C), perm_m)
  return out


# speedup vs baseline: 12.6864x; 1.0340x over previous
import functools

import jax
import jax.numpy as jnp
import numpy as np
from jax import lax
from jax.experimental import pallas as pl
from jax.experimental.pallas import tpu as pltpu
from jax.experimental.pallas import tpu_sc as plsc

N = 10000
E = 320000
D = 128
H = 128
C = 64
G = 16

NC = 2
NS = 16
NW = NC * NS
CH = 64
EPW = 10240
NCHK = EPW // CH
SCCH = 16
NSC = NCHK // SCCH
TOTSC = NW * NSC
K0 = 11
K1 = 2 * NSC - K0
E_PAD = NW * EPW

_MESH = plsc.VectorSubcoreMesh(
    core_axis_name="c", subcore_axis_name="s", num_cores=NC, num_subcores=NS)


def _pack_cols(y):
  a = lax.bitcast_convert_type(y[:, :H // 2], jnp.int32)
  b = lax.bitcast_convert_type(y[:, H // 2:], jnp.int32)
  ra = a + jnp.int32(0x7FFF) + ((a >> 16) & 1)
  rb = b + jnp.int32(0x7FFF) + ((b >> 16) & 1)
  return (rb & jnp.int32(-65536)) | lax.shift_right_logical(ra, 16)


def _ew_body(srcf_ref, dstf_ref, attr_ref, p_ref, out_ref):
  s = srcf_ref[...]
  d = dstf_ref[...]
  t = attr_ref[...]
  h = jnp.maximum(s * p_ref[0] + d * p_ref[1] + t * p_ref[2] + p_ref[3], 0.0)
  w = jax.nn.sigmoid(h * p_ref[4] + p_ref[5])
  r = lax.broadcasted_iota(jnp.int32, w.shape, 0)
  out_ref[...] = jnp.where(r < E // 128, w, 0.0)


def _edge_weights(srcf, dstf, attr, params):
  out = pl.pallas_call(
      _ew_body,
      out_shape=jax.ShapeDtypeStruct((E_PAD // 128, 128), jnp.float32),
      in_specs=[
          pl.BlockSpec(memory_space=pltpu.VMEM),
          pl.BlockSpec(memory_space=pltpu.VMEM),
          pl.BlockSpec(memory_space=pltpu.VMEM),
          pl.BlockSpec(memory_space=pltpu.SMEM),
      ],
      out_specs=pl.BlockSpec(memory_space=pltpu.VMEM),
  )(srcf, dstf, attr, params)
  return out


def _dis_from(degp_ref):
  deg = degp_ref[:, 0:1] + degp_ref[:, 1:2] + 1.0
  return lax.rsqrt(jnp.maximum(deg, 1e-12))


def _mm_body(x_ref, w_ref, out_ref):
  out_ref[...] = jnp.dot(x_ref[...], w_ref[...],
                         preferred_element_type=jnp.float32)


def _scale_body(degp_ref, h_ref, out_ref, pk_ref):
  ht = _dis_from(degp_ref) * h_ref[...]
  out_ref[...] = ht
  pk_ref[...] = _pack_cols(ht)


def _h2_body(degp_ref, a0_ref, a1_ref, ht_ref, w_ref, b_ref, out_ref,
             pk_ref):
  dis = _dis_from(degp_ref)
  agg = a0_ref[...] + a1_ref[...]
  z = jnp.maximum(dis * (agg + ht_ref[...]) + b_ref[...], 0.0)
  ht2 = dis * jnp.dot(z, w_ref[...], preferred_element_type=jnp.float32)
  out_ref[...] = ht2
  pk_ref[...] = _pack_cols(ht2)


def _out_body(degp_ref, a0_ref, a1_ref, ht_ref, b_ref, batch_ref, lw_ref,
              lb_ref, out_ref):
  dis = _dis_from(degp_ref)
  agg = a0_ref[...] + a1_ref[...]
  z = jnp.maximum(dis * (agg + ht_ref[...]) + b_ref[...], 0.0)
  gids = lax.broadcasted_iota(jnp.int32, (1, G), 1)
  onehot = (batch_ref[...] == gids).astype(jnp.float32)
  ze = jnp.concatenate([z, jnp.ones((N, 1), jnp.float32)], axis=1)
  sums = lax.dot_general(onehot, ze, (((0,), (0,)), ((), ())),
                         preferred_element_type=jnp.float32)
  pooled = sums[:, :H] / jnp.maximum(sums[:, H:H + 1], 1.0)
  out_ref[...] = jnp.dot(pooled, lw_ref[...],
                         preferred_element_type=jnp.float32) + lb_ref[...]


def _vmem_call(body, out_shape, *args):
  if isinstance(out_shape, tuple):
    out_specs = tuple(pl.BlockSpec(memory_space=pltpu.VMEM) for _ in out_shape)
  else:
    out_specs = pl.BlockSpec(memory_space=pltpu.VMEM)
  return pl.pallas_call(
      body,
      out_shape=out_shape,
      in_specs=[pl.BlockSpec(memory_space=pltpu.VMEM) for _ in args],
      out_specs=out_specs,
  )(*args)


def _deg_body(dst_hbm, ew_hbm, zeros_hbm, out_hbm, idx_v, w_v, acc_sh, sem):
  cid = lax.axis_index("c")
  sid = lax.axis_index("s")
  wid = sid * NC + cid
  pltpu.sync_copy(dst_hbm.at[wid], idx_v)
  pltpu.sync_copy(ew_hbm.at[wid], w_v)

  @pl.when(sid == 0)
  def _():
    pltpu.sync_copy(zeros_hbm, acc_sh)

  plsc.subcore_barrier()

  def chunk(j, carry):
    pltpu.sync_copy(w_v.at[j], acc_sh.at[idx_v.at[j]], add=True)
    return carry

  lax.fori_loop(0, NCHK, chunk, 0)
  plsc.subcore_barrier()

  @pl.when(sid == 0)
  def _():
    pltpu.sync_copy(acc_sh, out_hbm.at[cid])


@functools.partial(
    pl.kernel,
    out_type=jax.ShapeDtypeStruct((NC, N), jnp.float32),
    mesh=_MESH,
    scratch_types=[
        pltpu.VMEM((NCHK, CH), jnp.int32),
        pltpu.VMEM((NCHK, CH), jnp.float32),
        pltpu.VMEM_SHARED((N,), jnp.float32),
        pltpu.SemaphoreType.DMA,
    ],
)
def _deg_kernel(dst_hbm, ew_hbm, zeros_hbm, out_hbm, idx_v, w_v, acc_sh, sem):
  _deg_body(dst_hbm, ew_hbm, zeros_hbm, out_hbm, idx_v, w_v, acc_sh, sem)


def _agg_body(src_hbm, dst_hbm, ew_hbm, ht_hbm, zeros_hbm, out_hbm,
              src_v, dst_v, ew_v, buf0, buf1, sb0, sb1, sb2, sb3, acc_sh,
              sem0, sem1, ssem0, ssem1, ssem2, ssem3):
  cid = lax.axis_index("c")
  sid = lax.axis_index("s")
  base = jnp.where(cid == 0, sid * K0, NS * K0 + sid * K1)
  nsc_local = jnp.where(cid == 0, K0, K1)

  @pl.when(sid == 0)
  def _():
    pltpu.sync_copy(zeros_hbm, acc_sh)

  plsc.subcore_barrier()

  gbufs = (buf0, buf1)
  gsems = (sem0, sem1)
  sbufs = (sb0, sb1, sb2, sb3)
  ssems = (ssem0, ssem1, ssem2, ssem3)

  def unpack_scale(j, buf, sb):
    def scale(g, c2):
      ew16 = ew_v[j, pl.ds(g * 16, 16)]
      for k in range(16):
        s = ew16[k]
        e = g * 16 + k
        for q in range(H // 32):
          w16 = buf[e, pl.ds(q * 16, 16)]
          w32 = plsc.bitcast(w16, jnp.bfloat16)
          lo, hi = plsc.unpack(w32, format=plsc.PackFormat.INTERLEAVED)
          sb[e, pl.ds(q * 16, 16)] = lo * s
          sb[e, pl.ds(H // 2 + q * 16, 16)] = hi * s
      return c2

    lax.fori_loop(0, CH // 16, scale, 0)

  def super_body(s, carry):
    pltpu.sync_copy(src_hbm.at[base + s], src_v)
    pltpu.sync_copy(dst_hbm.at[base + s], dst_v)
    pltpu.sync_copy(ew_hbm.at[base + s], ew_v)
    pltpu.async_copy(ht_hbm.at[src_v.at[0]], buf0, sem0)

    def chunk4(jj, c2):
      for t in range(4):
        j = 4 * jj + t
        gb, gs = gbufs[(t + 1) % 2], gsems[(t + 1) % 2]
        cb, cs = gbufs[t % 2], gsems[t % 2]
        if t == 3:
          @pl.when(j + 1 < SCCH)
          def _(j=j, gb=gb, gs=gs):
            pltpu.async_copy(ht_hbm.at[src_v.at[j + 1]], gb, gs)
        else:
          pltpu.async_copy(ht_hbm.at[src_v.at[j + 1]], gb, gs)
        pltpu.make_async_copy(ht_hbm.at[src_v.at[j]], cb, cs).wait()

        @pl.when(jj >= 1)
        def _(t=t, j=j):
          pltpu.make_async_copy(
              sbufs[t], acc_sh.at[dst_v.at[j - 4]], ssems[t]).wait()

        unpack_scale(j, cb, sbufs[t])
        pltpu.async_copy(sbufs[t], acc_sh.at[dst_v.at[j]], ssems[t], add=True)
      return c2

    lax.fori_loop(0, SCCH // 4, chunk4, 0)
    for t in range(4):
      pltpu.make_async_copy(
          sbufs[t], acc_sh.at[dst_v.at[SCCH - 4 + t]], ssems[t]).wait()
    return carry

  lax.fori_loop(0, nsc_local, super_body, 0)
  plsc.subcore_barrier()

  @pl.when(sid == 0)
  def _():
    pltpu.sync_copy(acc_sh, out_hbm.at[cid])


@functools.partial(
    pl.kernel,
    out_type=jax.ShapeDtypeStruct((NC, N, H), jnp.float32),
    mesh=_MESH,
    scratch_types=[
        pltpu.VMEM((SCCH, CH), jnp.int32),
        pltpu.VMEM((SCCH, CH), jnp.int32),
        pltpu.VMEM((SCCH, CH), jnp.float32),
        pltpu.VMEM((CH, H // 2), jnp.int32),
        pltpu.VMEM((CH, H // 2), jnp.int32),
        pltpu.VMEM((CH, H), jnp.float32),
        pltpu.VMEM((CH, H), jnp.float32),
        pltpu.VMEM((CH, H), jnp.float32),
        pltpu.VMEM((CH, H), jnp.float32),
        pltpu.VMEM_SHARED((N, H), jnp.float32),
        pltpu.SemaphoreType.DMA,
        pltpu.SemaphoreType.DMA,
        pltpu.SemaphoreType.DMA,
        pltpu.SemaphoreType.DMA,
        pltpu.SemaphoreType.DMA,
        pltpu.SemaphoreType.DMA,
    ],
    compiler_params=pltpu.CompilerParams(
        needs_layout_passes=False, use_tc_tiling_on_sc=False),
)
def _agg_kernel(src_hbm, dst_hbm, ew_hbm, ht_hbm, zeros_hbm, out_hbm,
                src_v, dst_v, ew_v, buf0, buf1, sb0, sb1, sb2, sb3, acc_sh,
                sem0, sem1, ssem0, ssem1, ssem2, ssem3):
  _agg_body(src_hbm, dst_hbm, ew_hbm, ht_hbm, zeros_hbm, out_hbm,
            src_v, dst_v, ew_v, buf0, buf1, sb0, sb1, sb2, sb3, acc_sh,
            sem0, sem1, ssem0, ssem1, ssem2, ssem3)


def kernel(x, edge_index, edge_attr, batch, ee_w1, ee_b1, ee_w2, ee_b2,
           conv1_w, conv1_b, stem_w, stem_b, lin_w, lin_b):
  src = edge_index[0]
  dst = edge_index[1]
  pad = E_PAD - E
  src_p = jnp.pad(src, (0, pad))
  dst_p = jnp.pad(dst, (0, pad))
  attr_p = jnp.pad(edge_attr[:, 0], (0, pad))

  params = jnp.stack([
      ee_w1[0, 0], ee_w1[1, 0], ee_w1[2, 0], ee_b1[0], ee_w2[0, 0], ee_b2[0],
  ])

  srcf = src_p.astype(jnp.float32).reshape(E_PAD // 128, 128)
  dstf = dst_p.astype(jnp.float32).reshape(E_PAD // 128, 128)
  attrf = attr_p.reshape(E_PAD // 128, 128)
  ew = _edge_weights(srcf, dstf, attrf, params)

  src3 = src_p.reshape(NW, NCHK, CH)
  dst3 = dst_p.reshape(NW, NCHK, CH)
  ew3 = ew.reshape(NW, NCHK, CH)
  src4 = src_p.reshape(TOTSC, SCCH, CH)
  dst4 = dst_p.reshape(TOTSC, SCCH, CH)
  ew4 = ew.reshape(TOTSC, SCCH, CH)

  zeros_n = jnp.zeros((N,), jnp.float32)
  zeros_nh = jnp.zeros((N, H), jnp.float32)

  degp = _deg_kernel(dst3, ew3, zeros_n)
  degp_t = degp.T

  fnh = jax.ShapeDtypeStruct((N, H), jnp.float32)
  inh2 = jax.ShapeDtypeStruct((N, H // 2), jnp.int32)

  h1 = _vmem_call(_mm_body, fnh, x, conv1_w)
  ht1, pk1 = _vmem_call(_scale_body, (fnh, inh2), degp_t, h1)
  agg1 = _agg_kernel(src4, dst4, ew4, pk1, zeros_nh)

  ht2, pk2 = _vmem_call(_h2_body, (fnh, inh2),
                        degp_t, agg1[0], agg1[1], ht1, stem_w,
                        conv1_b.reshape(1, H))
  agg2 = _agg_kernel(src4, dst4, ew4, pk2, zeros_nh)

  out = _vmem_call(_out_body, jax.ShapeDtypeStruct((G, C), jnp.float32),
                   degp_t, agg2[0], agg2[1], ht2, stem_b.reshape(1, H),
                   batch.reshape(N, 1), lin_w, lin_b.reshape(1, C))
  return out


# speedup vs baseline: 13.0962x; 1.0323x over previous
import functools

import jax
import jax.numpy as jnp
import numpy as np
from jax import lax
from jax.experimental import pallas as pl
from jax.experimental.pallas import tpu as pltpu
from jax.experimental.pallas import tpu_sc as plsc

N = 10000
E = 320000
D = 128
H = 128
C = 64
G = 16

NC = 2
NS = 16
NW = NC * NS
CH = 64
EPW = 10240
NCHK = EPW // CH
SCCH = 16
NSC = NCHK // SCCH
TOTSC = NW * NSC
K0 = 11
K1 = 2 * NSC - K0
E_PAD = NW * EPW

_MESH = plsc.VectorSubcoreMesh(
    core_axis_name="c", subcore_axis_name="s", num_cores=NC, num_subcores=NS)


def _pack_cols(y):
  a = lax.bitcast_convert_type(y[:, :H // 2], jnp.int32)
  b = lax.bitcast_convert_type(y[:, H // 2:], jnp.int32)
  ra = a + jnp.int32(0x7FFF) + ((a >> 16) & 1)
  rb = b + jnp.int32(0x7FFF) + ((b >> 16) & 1)
  return (rb & jnp.int32(-65536)) | lax.shift_right_logical(ra, 16)


def _ew_body(srcf_ref, dstf_ref, attr_ref, p_ref, out_ref):
  s = srcf_ref[...]
  d = dstf_ref[...]
  t = attr_ref[...]
  h = jnp.maximum(s * p_ref[0] + d * p_ref[1] + t * p_ref[2] + p_ref[3], 0.0)
  w = jax.nn.sigmoid(h * p_ref[4] + p_ref[5])
  r = lax.broadcasted_iota(jnp.int32, w.shape, 0)
  out_ref[...] = jnp.where(r < E // 128, w, 0.0)


def _edge_weights(srcf, dstf, attr, params):
  out = pl.pallas_call(
      _ew_body,
      out_shape=jax.ShapeDtypeStruct((E_PAD // 128, 128), jnp.float32),
      in_specs=[
          pl.BlockSpec(memory_space=pltpu.VMEM),
          pl.BlockSpec(memory_space=pltpu.VMEM),
          pl.BlockSpec(memory_space=pltpu.VMEM),
          pl.BlockSpec(memory_space=pltpu.SMEM),
      ],
      out_specs=pl.BlockSpec(memory_space=pltpu.VMEM),
  )(srcf, dstf, attr, params)
  return out


def _dis_from(degp_ref):
  deg = degp_ref[:, 0:1] + degp_ref[:, 1:2] + 1.0
  return lax.rsqrt(jnp.maximum(deg, 1e-12))


def _mm_body(x_ref, w_ref, out_ref):
  out_ref[...] = jnp.dot(x_ref[...], w_ref[...],
                         preferred_element_type=jnp.float32)


def _scale_body(degp_ref, h_ref, out_ref, pk_ref):
  ht = _dis_from(degp_ref) * h_ref[...]
  out_ref[...] = ht
  pk_ref[...] = _pack_cols(ht)


def _h2_body(degp_ref, a0_ref, a1_ref, ht_ref, w_ref, b_ref, out_ref,
             pk_ref):
  dis = _dis_from(degp_ref)
  agg = a0_ref[...] + a1_ref[...]
  z = jnp.maximum(dis * (agg + ht_ref[...]) + b_ref[...], 0.0)
  ht2 = dis * jnp.dot(z, w_ref[...], preferred_element_type=jnp.float32)
  out_ref[...] = ht2
  pk_ref[...] = _pack_cols(ht2)


def _out_body(degp_ref, a0_ref, a1_ref, ht_ref, b_ref, batch_ref, lw_ref,
              lb_ref, out_ref):
  dis = _dis_from(degp_ref)
  agg = a0_ref[...] + a1_ref[...]
  z = jnp.maximum(dis * (agg + ht_ref[...]) + b_ref[...], 0.0)
  gids = lax.broadcasted_iota(jnp.int32, (1, G), 1)
  onehot = (batch_ref[...] == gids).astype(jnp.float32)
  ze = jnp.concatenate([z, jnp.ones((N, 1), jnp.float32)], axis=1)
  sums = lax.dot_general(onehot, ze, (((0,), (0,)), ((), ())),
                         preferred_element_type=jnp.float32)
  pooled = sums[:, :H] / jnp.maximum(sums[:, H:H + 1], 1.0)
  out_ref[...] = jnp.dot(pooled, lw_ref[...],
                         preferred_element_type=jnp.float32) + lb_ref[...]


def _vmem_call(body, out_shape, *args):
  if isinstance(out_shape, tuple):
    out_specs = tuple(pl.BlockSpec(memory_space=pltpu.VMEM) for _ in out_shape)
  else:
    out_specs = pl.BlockSpec(memory_space=pltpu.VMEM)
  return pl.pallas_call(
      body,
      out_shape=out_shape,
      in_specs=[pl.BlockSpec(memory_space=pltpu.VMEM) for _ in args],
      out_specs=out_specs,
  )(*args)


def _deg_body(dst_hbm, ew_hbm, zeros_hbm, out_hbm, idx_v, w_v, acc_sh, sem):
  cid = lax.axis_index("c")
  sid = lax.axis_index("s")
  wid = sid * NC + cid
  pltpu.sync_copy(dst_hbm.at[wid], idx_v)
  pltpu.sync_copy(ew_hbm.at[wid], w_v)

  @pl.when(sid == 0)
  def _():
    pltpu.sync_copy(zeros_hbm, acc_sh)

  plsc.subcore_barrier()

  def chunk(j, carry):
    pltpu.sync_copy(w_v.at[j], acc_sh.at[idx_v.at[j]], add=True)
    return carry

  lax.fori_loop(0, NCHK, chunk, 0)
  plsc.subcore_barrier()

  @pl.when(sid == 0)
  def _():
    pltpu.sync_copy(acc_sh, out_hbm.at[cid])


@functools.partial(
    pl.kernel,
    out_type=jax.ShapeDtypeStruct((NC, N), jnp.float32),
    mesh=_MESH,
    scratch_types=[
        pltpu.VMEM((NCHK, CH), jnp.int32),
        pltpu.VMEM((NCHK, CH), jnp.float32),
        pltpu.VMEM_SHARED((N,), jnp.float32),
        pltpu.SemaphoreType.DMA,
    ],
)
def _deg_kernel(dst_hbm, ew_hbm, zeros_hbm, out_hbm, idx_v, w_v, acc_sh, sem):
  _deg_body(dst_hbm, ew_hbm, zeros_hbm, out_hbm, idx_v, w_v, acc_sh, sem)


def _agg_body(src_hbm, dst_hbm, ew_hbm, ht_hbm, zeros_hbm, out_hbm,
              src_v0, dst_v0, ew_v0, src_v1, dst_v1, ew_v1,
              buf0, buf1, buf2, buf3, sb0, sb1, acc_sh,
              sem0, sem1, sem2, sem3, ssem0, ssem1, stsem0, stsem1):
  cid = lax.axis_index("c")
  sid = lax.axis_index("s")
  base = jnp.where(cid == 0, sid * K0, NS * K0 + sid * K1)
  nsc_local = jnp.where(cid == 0, K0, K1)

  @pl.when(sid == 0)
  def _():
    pltpu.sync_copy(zeros_hbm, acc_sh)

  plsc.subcore_barrier()

  gbufs = (buf0, buf1, buf2, buf3)
  gsems = (sem0, sem1, sem2, sem3)
  sbufs = (sb0, sb1)
  ssems = (ssem0, ssem1)
  stage = ((src_v0, dst_v0, ew_v0), (src_v1, dst_v1, ew_v1))
  stsems = (stsem0, stsem1)

  def unpack_scale(j, ew_v, buf, sb):
    def scale(g, c2):
      ew16 = ew_v[j, pl.ds(g * 16, 16)]
      for k in range(16):
        s = ew16[k]
        e = g * 16 + k
        for q in range(H // 32):
          w16 = buf[e, pl.ds(q * 16, 16)]
          w32 = plsc.bitcast(w16, jnp.bfloat16)
          lo, hi = plsc.unpack(w32, format=plsc.PackFormat.INTERLEAVED)
          sb[e, pl.ds(q * 16, 16)] = lo * s
          sb[e, pl.ds(H // 2 + q * 16, 16)] = hi * s
      return c2

    lax.fori_loop(0, CH // 16, scale, 0)

  def stage_start(s, slot):
    src_v, dst_v, ew_v = stage[slot]
    pltpu.async_copy(src_hbm.at[base + s], src_v, stsems[slot])
    pltpu.async_copy(dst_hbm.at[base + s], dst_v, stsems[slot])
    pltpu.async_copy(ew_hbm.at[base + s], ew_v, stsems[slot])

  def stage_wait(s, slot):
    src_v, dst_v, ew_v = stage[slot]
    pltpu.make_async_copy(src_hbm.at[base + s], src_v, stsems[slot]).wait()
    pltpu.make_async_copy(dst_hbm.at[base + s], dst_v, stsems[slot]).wait()
    pltpu.make_async_copy(ew_hbm.at[base + s], ew_v, stsems[slot]).wait()

  def run_super(s, slot):
    src_v, dst_v, ew_v = stage[slot]
    stage_wait(s, slot)
    for c in range(3):
      pltpu.async_copy(ht_hbm.at[src_v.at[c]], gbufs[c], gsems[c])

    def chunk4(jj, c2):
      for t in range(4):
        j = 4 * jj + t
        pltpu.make_async_copy(
            ht_hbm.at[src_v.at[j]], gbufs[t], gsems[t]).wait()

        @pl.when(j + 3 < SCCH)
        def _(j=j, t=t):
          pltpu.async_copy(
              ht_hbm.at[src_v.at[j + 3]], gbufs[(t + 3) % 4],
              gsems[(t + 3) % 4])

        if t < 2:
          @pl.when(jj >= 1)
          def _(t=t, j=j):
            pltpu.make_async_copy(
                sbufs[t], acc_sh.at[dst_v.at[j - 2]], ssems[t]).wait()
        else:
          pltpu.make_async_copy(
              sbufs[t % 2], acc_sh.at[dst_v.at[j - 2]], ssems[t % 2]).wait()
        unpack_scale(j, ew_v, gbufs[t], sbufs[t % 2])
        pltpu.async_copy(
            sbufs[t % 2], acc_sh.at[dst_v.at[j]], ssems[t % 2], add=True)
      return c2

    lax.fori_loop(0, SCCH // 4, chunk4, 0)
    for t in range(2):
      pltpu.make_async_copy(
          sbufs[t], acc_sh.at[dst_v.at[SCCH - 2 + t]], ssems[t]).wait()
    @pl.when(s + 2 < nsc_local)
    def _():
      stage_start(s + 2, slot)

  stage_start(0, 0)

  @pl.when(nsc_local >= 2)
  def _():
    stage_start(1, 1)

  def super_pair(ss, carry):
    run_super(2 * ss, 0)
    run_super(2 * ss + 1, 1)
    return carry

  lax.fori_loop(0, nsc_local // 2, super_pair, 0)

  @pl.when(nsc_local % 2 == 1)
  def _():
    run_super(nsc_local - 1, 0)

  plsc.subcore_barrier()

  @pl.when(sid == 0)
  def _():
    pltpu.sync_copy(acc_sh, out_hbm.at[cid])


@functools.partial(
    pl.kernel,
    out_type=jax.ShapeDtypeStruct((NC, N, H), jnp.float32),
    mesh=_MESH,
    scratch_types=[
        pltpu.VMEM((SCCH, CH), jnp.int32),
        pltpu.VMEM((SCCH, CH), jnp.int32),
        pltpu.VMEM((SCCH, CH), jnp.float32),
        pltpu.VMEM((SCCH, CH), jnp.int32),
        pltpu.VMEM((SCCH, CH), jnp.int32),
        pltpu.VMEM((SCCH, CH), jnp.float32),
        pltpu.VMEM((CH, H // 2), jnp.int32),
        pltpu.VMEM((CH, H // 2), jnp.int32),
        pltpu.VMEM((CH, H // 2), jnp.int32),
        pltpu.VMEM((CH, H // 2), jnp.int32),
        pltpu.VMEM((CH, H), jnp.float32),
        pltpu.VMEM((CH, H), jnp.float32),
        pltpu.VMEM_SHARED((N, H), jnp.float32),
        pltpu.SemaphoreType.DMA,
        pltpu.SemaphoreType.DMA,
        pltpu.SemaphoreType.DMA,
        pltpu.SemaphoreType.DMA,
        pltpu.SemaphoreType.DMA,
        pltpu.SemaphoreType.DMA,
        pltpu.SemaphoreType.DMA,
        pltpu.SemaphoreType.DMA,
    ],
    compiler_params=pltpu.CompilerParams(
        needs_layout_passes=False, use_tc_tiling_on_sc=False),
)
def _agg_kernel(src_hbm, dst_hbm, ew_hbm, ht_hbm, zeros_hbm, out_hbm,
                src_v0, dst_v0, ew_v0, src_v1, dst_v1, ew_v1,
                buf0, buf1, buf2, buf3, sb0, sb1, acc_sh,
                sem0, sem1, sem2, sem3, ssem0, ssem1, stsem0, stsem1):
  _agg_body(src_hbm, dst_hbm, ew_hbm, ht_hbm, zeros_hbm, out_hbm,
            src_v0, dst_v0, ew_v0, src_v1, dst_v1, ew_v1,
            buf0, buf1, buf2, buf3, sb0, sb1, acc_sh,
            sem0, sem1, sem2, sem3, ssem0, ssem1, stsem0, stsem1)


def kernel(x, edge_index, edge_attr, batch, ee_w1, ee_b1, ee_w2, ee_b2,
           conv1_w, conv1_b, stem_w, stem_b, lin_w, lin_b):
  src = edge_index[0]
  dst = edge_index[1]
  pad = E_PAD - E
  src_p = jnp.pad(src, (0, pad))
  dst_p = jnp.pad(dst, (0, pad))
  attr_p = jnp.pad(edge_attr[:, 0], (0, pad))

  params = jnp.stack([
      ee_w1[0, 0], ee_w1[1, 0], ee_w1[2, 0], ee_b1[0], ee_w2[0, 0], ee_b2[0],
  ])

  srcf = src_p.astype(jnp.float32).reshape(E_PAD // 128, 128)
  dstf = dst_p.astype(jnp.float32).reshape(E_PAD // 128, 128)
  attrf = attr_p.reshape(E_PAD // 128, 128)
  ew = _edge_weights(srcf, dstf, attrf, params)

  src3 = src_p.reshape(NW, NCHK, CH)
  dst3 = dst_p.reshape(NW, NCHK, CH)
  ew3 = ew.reshape(NW, NCHK, CH)
  src4 = src_p.reshape(TOTSC, SCCH, CH)
  dst4 = dst_p.reshape(TOTSC, SCCH, CH)
  ew4 = ew.reshape(TOTSC, SCCH, CH)

  zeros_n = jnp.zeros((N,), jnp.float32)
  zeros_nh = jnp.zeros((N, H), jnp.float32)

  degp = _deg_kernel(dst3, ew3, zeros_n)
  degp_t = degp.T

  fnh = jax.ShapeDtypeStruct((N, H), jnp.float32)
  inh2 = jax.ShapeDtypeStruct((N, H // 2), jnp.int32)

  h1 = _vmem_call(_mm_body, fnh, x, conv1_w)
  ht1, pk1 = _vmem_call(_scale_body, (fnh, inh2), degp_t, h1)
  agg1 = _agg_kernel(src4, dst4, ew4, pk1, zeros_nh)

  ht2, pk2 = _vmem_call(_h2_body, (fnh, inh2),
                        degp_t, agg1[0], agg1[1], ht1, stem_w,
                        conv1_b.reshape(1, H))
  agg2 = _agg_kernel(src4, dst4, ew4, pk2, zeros_nh)

  out = _vmem_call(_out_body, jax.ShapeDtypeStruct((G, C), jnp.float32),
                   degp_t, agg2[0], agg2[1], ht2, stem_b.reshape(1, H),
                   batch.reshape(N, 1), lin_w, lin_b.reshape(1, C))
  return out
